# baseline (speedup 1.0000x reference)
import jax
import jax.numpy as jnp
from jax import lax
from jax.experimental import pallas as pl
from jax.experimental.pallas import tpu as pltpu
from jax.experimental.pallas import tpu_sc as plsc

_NUM_EVENT = 100000
_NUM_STATION = 100
_B = 16384
_REG = 0.001

_NC = 2
_NS = 16
_NW = _NC * _NS
_CHUNK = _B // _NW
_GCHUNK = 128
_NG = _CHUNK // _GCHUNK
_L = 16


def _rsqrt(x):
    i = plsc.bitcast(x, jnp.int32)
    i = jnp.int32(0x5F3759DF) - (i >> 1)
    y = plsc.bitcast(i, jnp.float32)
    for _ in range(4):
        y = y * (1.5 - 0.5 * x * y * y)
    return y


def _sc_body(evx_hbm, evy_hbm, evz_hbm, evt_hbm, stloc_hbm, stdt_hbm,
             pt_hbm, pw_hbm, sidx_hbm, eidx_hbm, t_hbm, parts_hbm,
             eidx_v, sidx_v, ex_v, ey_v, ez_v, et_v, pt_v, pw_v, t_v,
             stloc_v, stdt_v, part_v, sem):
    wid = lax.axis_index("s") * _NC + lax.axis_index("c")
    base = wid * _CHUNK
    bsl = pl.ds(base, _CHUNK)

    pltpu.sync_copy(eidx_hbm.at[bsl], eidx_v)
    descs = []
    for k in range(_NG):
        isl = pl.ds(k * _GCHUNK, _GCHUNK)
        idx = eidx_v.at[isl]
        descs.append(pltpu.async_copy(evx_hbm.at[idx], ex_v.at[isl], sem))
        descs.append(pltpu.async_copy(evy_hbm.at[idx], ey_v.at[isl], sem))
        descs.append(pltpu.async_copy(evz_hbm.at[idx], ez_v.at[isl], sem))
        descs.append(pltpu.async_copy(evt_hbm.at[idx], et_v.at[isl], sem))

    pltpu.sync_copy(sidx_hbm.at[bsl], sidx_v)
    pltpu.sync_copy(pt_hbm.at[bsl], pt_v)
    pltpu.sync_copy(pw_hbm.at[bsl], pw_v)
    pltpu.sync_copy(stloc_hbm, stloc_v)
    pltpu.sync_copy(stdt_hbm, stdt_v)
    for d in descs:
        d.wait()

    lane = lax.iota(jnp.int32, _L)
    zero = jnp.zeros((_L,), jnp.float32)
    zero_i = jnp.zeros((_L,), jnp.int32)

    def chunk(j, carry):
        hacc, sacc = carry
        off = j * _L
        sl = pl.ds(off, _L)
        row = off + lane
        sidx = sidx_v[sl]
        elx = ex_v[sl]
        ely = ey_v[sl]
        elz = ez_v[sl]
        et = plsc.load_gather(et_v, [row, zero_i])
        pt = plsc.load_gather(pt_v, [row, zero_i])
        pw = plsc.load_gather(pw_v, [row, zero_i])
        s3 = sidx * 3
        slx = plsc.load_gather(stloc_v, [sidx, zero_i])
        sly = plsc.load_gather(stloc_v, [sidx, zero_i + 1])
        slz = plsc.load_gather(stloc_v, [sidx, zero_i + 2])
        sdt = plsc.load_gather(stdt_v, [sidx, zero_i])
        del s3
        dx = elx - slx
        dy = ely - sly
        dz = elz - slz
        s = dx * dx + dy * dy + dz * dz
        dist = s * _rsqrt(s)
        t = et + dist + sdt
        plsc.store_scatter(t_v, [row, zero_i], t)
        err = t - pt
        ae = jnp.abs(err)
        hub = jnp.where(ae < 1.0, 0.5 * err * err, ae - 0.5)
        return hacc + hub * pw, sacc + jnp.abs(sdt)

    hacc, sacc = lax.fori_loop(0, _CHUNK // _L, chunk, (zero, zero))
    hsum = jnp.sum(hacc)
    ssum = jnp.sum(sacc)
    part_v[...] = jnp.where(lane == 0, hsum, jnp.where(lane == 1, ssum, 0.0))

    pltpu.sync_copy(t_v, t_hbm.at[bsl])
    pltpu.sync_copy(part_v, parts_hbm.at[wid])


@jax.jit
def _run(evx, evy, evz, evt, stloc, stdt, pt, pw, sidx, eidx):
    mesh = plsc.VectorSubcoreMesh(core_axis_name="c", subcore_axis_name="s",
                                  num_cores=_NC, num_subcores=_NS)
    f = pl.kernel(
        _sc_body,
        out_type=(
            jax.ShapeDtypeStruct((_B, 1), jnp.float32),
            jax.ShapeDtypeStruct((_NW, _L), jnp.float32),
        ),
        mesh=mesh,
        scratch_types=[
            pltpu.VMEM((_CHUNK,), jnp.int32),
            pltpu.VMEM((_CHUNK,), jnp.int32),
            pltpu.VMEM((_CHUNK,), jnp.float32),
            pltpu.VMEM((_CHUNK,), jnp.float32),
            pltpu.VMEM((_CHUNK,), jnp.float32),
            pltpu.VMEM((_CHUNK, 1), jnp.float32),
            pltpu.VMEM((_CHUNK, 1), jnp.float32),
            pltpu.VMEM((_CHUNK, 1), jnp.float32),
            pltpu.VMEM((_CHUNK, 1), jnp.float32),
            pltpu.VMEM((_NUM_STATION, 3), jnp.float32),
            pltpu.VMEM((_NUM_STATION, 1), jnp.float32),
            pltpu.VMEM((_L,), jnp.float32),
            pltpu.SemaphoreType.DMA,
        ],
        compiler_params=pltpu.CompilerParams(needs_layout_passes=False,
                                             use_tc_tiling_on_sc=False),
        name="travel_time_sc",
    )
    t, parts = f(evx, evy, evz, evt, stloc, stdt, pt, pw, sidx, eidx)
    loss = (jnp.sum(parts[:, 0]) + _REG * jnp.sum(parts[:, 1])) / _B
    return t, loss


def kernel(event_loc_w, event_time_w, station_dt_w, station_loc, phase_time,
           phase_weight, station_index, event_index):
    return _run(
        event_loc_w[:, 0],
        event_loc_w[:, 1],
        event_loc_w[:, 2],
        event_time_w,
        station_loc,
        station_dt_w,
        phase_time,
        phase_weight,
        station_index,
        event_index,
    )


# speedup vs baseline: 4.7598x; 4.7598x over previous
import jax
import jax.numpy as jnp
from jax import lax
from jax.experimental import pallas as pl
from jax.experimental.pallas import tpu as pltpu
from jax.experimental.pallas import tpu_sc as plsc

_NUM_EVENT = 100000
_NUM_STATION = 100
_B = 16384
_REG = 0.001

_NC = 2
_NS = 16
_NW = _NC * _NS
_CHUNK = _B // _NW
_GCHUNK = 512
_NG = _CHUNK // _GCHUNK
_L = 16


def _rsqrt(x):
    i = plsc.bitcast(x, jnp.int32)
    i = jnp.int32(0x5F3759DF) - (i >> 1)
    y = plsc.bitcast(i, jnp.float32)
    for _ in range(4):
        y = y * (1.5 - 0.5 * x * y * y)
    return y


def _sc_body(evx_hbm, evy_hbm, evz_hbm, evt_hbm, stloc_hbm, stdt_hbm,
             pt_hbm, pw_hbm, sidx_hbm, eidx_hbm, t_hbm, parts_hbm,
             eidx_v, sidx_v, ex_v, ey_v, ez_v, et_v, pt_v, pw_v, t_v,
             stloc_v, stdt_v, part_v, sem):
    wid = lax.axis_index("s") * _NC + lax.axis_index("c")
    base = wid * _CHUNK

    pltpu.sync_copy(eidx_hbm.at[pl.ds(base, _CHUNK)], eidx_v)
    descs = []
    for k in range(_NG):
        isl = pl.ds(k * _GCHUNK, _GCHUNK)
        idx = eidx_v.at[isl]
        descs.append(pltpu.async_copy(evx_hbm.at[idx], ex_v.at[isl], sem))
        descs.append(pltpu.async_copy(evy_hbm.at[idx], ey_v.at[isl], sem))
        descs.append(pltpu.async_copy(evz_hbm.at[idx], ez_v.at[isl], sem))
        descs.append(pltpu.async_copy(evt_hbm.at[idx], et_v.at[isl], sem))

    pltpu.sync_copy(sidx_hbm.at[pl.ds(base, _CHUNK)], sidx_v)
    pltpu.sync_copy(pt_hbm.at[pl.ds(base, _CHUNK)], pt_v)
    pltpu.sync_copy(pw_hbm.at[pl.ds(base, _CHUNK)], pw_v)
    pltpu.sync_copy(stloc_hbm, stloc_v)
    pltpu.sync_copy(stdt_hbm, stdt_v)

    lane = lax.iota(jnp.int32, _L)
    zero = jnp.zeros((_L,), jnp.float32)

    def chunk(j, carry):
        hacc, sacc = carry
        off = j * _L
        sl = pl.ds(off, _L)
        sidx = sidx_v[sl]
        elx = ex_v[sl]
        ely = ey_v[sl]
        elz = ez_v[sl]
        et = et_v[sl]
        pt = pt_v[sl]
        pw = pw_v[sl]
        s3 = sidx * 3
        slx = plsc.load_gather(stloc_v, [s3])
        sly = plsc.load_gather(stloc_v, [s3 + 1])
        slz = plsc.load_gather(stloc_v, [s3 + 2])
        sdt = plsc.load_gather(stdt_v, [sidx])
        dx = elx - slx
        dy = ely - sly
        dz = elz - slz
        s = dx * dx + dy * dy + dz * dz
        dist = s * _rsqrt(s)
        t = et + dist + sdt
        t_v[sl] = t
        err = t - pt
        ae = jnp.abs(err)
        hub = jnp.where(ae < 1.0, 0.5 * err * err, ae - 0.5)
        return hacc + hub * pw, sacc + jnp.abs(sdt)

    hacc, sacc = zero, zero
    for k in range(_NG):
        for d in descs[4 * k:4 * k + 4]:
            d.wait()
        hacc, sacc = lax.fori_loop(
            k * (_GCHUNK // _L), (k + 1) * (_GCHUNK // _L), chunk,
            (hacc, sacc))

    hsum = jnp.sum(hacc)
    ssum = jnp.sum(sacc)
    part_v[...] = jnp.where(lane == 0, hsum, jnp.where(lane == 1, ssum, 0.0))

    pltpu.sync_copy(t_v, t_hbm.at[pl.ds(base, _CHUNK)])
    pltpu.sync_copy(part_v, parts_hbm.at[wid])


@jax.jit
def _run(evx, evy, evz, evt, stloc, stdt, pt, pw, sidx, eidx):
    mesh = plsc.VectorSubcoreMesh(core_axis_name="c", subcore_axis_name="s",
                                  num_cores=_NC, num_subcores=_NS)
    f = pl.kernel(
        _sc_body,
        out_type=(
            jax.ShapeDtypeStruct((_B,), jnp.float32),
            jax.ShapeDtypeStruct((_NW, _L), jnp.float32),
        ),
        mesh=mesh,
        scratch_types=[
            pltpu.VMEM((_CHUNK,), jnp.int32),
            pltpu.VMEM((_CHUNK,), jnp.int32),
            pltpu.VMEM((_CHUNK,), jnp.float32),
            pltpu.VMEM((_CHUNK,), jnp.float32),
            pltpu.VMEM((_CHUNK,), jnp.float32),
            pltpu.VMEM((_CHUNK,), jnp.float32),
            pltpu.VMEM((_CHUNK,), jnp.float32),
            pltpu.VMEM((_CHUNK,), jnp.float32),
            pltpu.VMEM((_CHUNK,), jnp.float32),
            pltpu.VMEM((_NUM_STATION * 3,), jnp.float32),
            pltpu.VMEM((_NUM_STATION,), jnp.float32),
            pltpu.VMEM((_L,), jnp.float32),
            pltpu.SemaphoreType.DMA,
        ],
        compiler_params=pltpu.CompilerParams(needs_layout_passes=False,
                                             use_tc_tiling_on_sc=False),
        name="travel_time_sc",
    )
    t_flat, parts = f(evx, evy, evz, evt, stloc, stdt, pt, pw, sidx, eidx)
    t = t_flat.reshape(_B, 1)
    loss = (jnp.sum(parts[:, 0]) + _REG * jnp.sum(parts[:, 1])) / _B
    return t, loss


def kernel(event_loc_w, event_time_w, station_dt_w, station_loc, phase_time,
           phase_weight, station_index, event_index):
    return _run(
        event_loc_w[:, 0],
        event_loc_w[:, 1],
        event_loc_w[:, 2],
        event_time_w.reshape(-1),
        station_loc.reshape(-1),
        station_dt_w.reshape(-1),
        phase_time.reshape(-1),
        phase_weight.reshape(-1),
        station_index,
        event_index,
    )


# speedup vs baseline: 5.2733x; 1.1079x over previous
import jax
import jax.numpy as jnp
from jax import lax
from jax.experimental import pallas as pl
from jax.experimental.pallas import tpu as pltpu
from jax.experimental.pallas import tpu_sc as plsc

_NUM_EVENT = 100000
_NUM_STATION = 100
_B = 16384
_REG = 0.001

_NC = 2
_NS = 16
_NW = _NC * _NS
_CHUNK = _B // _NW
_GCHUNK = 512
_NG = _CHUNK // _GCHUNK
_L = 16


def _rsqrt(x):
    i = plsc.bitcast(x, jnp.int32)
    i = jnp.int32(0x5F3759DF) - (i >> 1)
    y = plsc.bitcast(i, jnp.float32)
    for _ in range(4):
        y = y * (1.5 - 0.5 * x * y * y)
    return y


def _sc_body(evp_hbm, evt_hbm, stloc_hbm, stdt_hbm,
             pt_hbm, pw_hbm, sidx_hbm, eidx_hbm, t_hbm, parts_hbm,
             eidx_v, sidx_v, ex_v, ey_v, ez_v, et_v, pt_v, pw_v, t_v,
             stloc_v, stdt_v, part_v, sem):
    wid = lax.axis_index("s") * _NC + lax.axis_index("c")
    base = wid * _CHUNK

    pltpu.sync_copy(eidx_hbm.at[pl.ds(base, _CHUNK)], eidx_v)
    descs = []
    for k in range(_NG):
        isl = pl.ds(k * _GCHUNK, _GCHUNK)
        idx = eidx_v.at[isl]
        descs.append(pltpu.async_copy(
            evp_hbm.at[pl.ds(0, _NUM_EVENT)].at[idx], ex_v.at[isl], sem))
        descs.append(pltpu.async_copy(
            evp_hbm.at[pl.ds(_NUM_EVENT, _NUM_EVENT)].at[idx],
            ey_v.at[isl], sem))
        descs.append(pltpu.async_copy(
            evp_hbm.at[pl.ds(2 * _NUM_EVENT, _NUM_EVENT)].at[idx],
            ez_v.at[isl], sem))
        descs.append(pltpu.async_copy(evt_hbm.at[idx], et_v.at[isl], sem))

    pltpu.sync_copy(sidx_hbm.at[pl.ds(base, _CHUNK)], sidx_v)
    pltpu.sync_copy(pt_hbm.at[pl.ds(base, _CHUNK)], pt_v)
    pltpu.sync_copy(pw_hbm.at[pl.ds(base, _CHUNK)], pw_v)
    pltpu.sync_copy(stloc_hbm, stloc_v)
    pltpu.sync_copy(stdt_hbm, stdt_v)

    lane = lax.iota(jnp.int32, _L)
    zero = jnp.zeros((_L,), jnp.float32)

    def chunk(j, carry):
        hacc, sacc = carry
        off = j * _L
        sl = pl.ds(off, _L)
        sidx = sidx_v[sl]
        elx = ex_v[sl]
        ely = ey_v[sl]
        elz = ez_v[sl]
        et = et_v[sl]
        pt = pt_v[sl]
        pw = pw_v[sl]
        s3 = sidx * 3
        slx = plsc.load_gather(stloc_v, [s3])
        sly = plsc.load_gather(stloc_v, [s3 + 1])
        slz = plsc.load_gather(stloc_v, [s3 + 2])
        sdt = plsc.load_gather(stdt_v, [sidx])
        dx = elx - slx
        dy = ely - sly
        dz = elz - slz
        s = dx * dx + dy * dy + dz * dz
        dist = s * _rsqrt(s)
        t = et + dist + sdt
        t_v[sl] = t
        err = t - pt
        ae = jnp.abs(err)
        hub = jnp.where(ae < 1.0, 0.5 * err * err, ae - 0.5)
        return hacc + hub * pw, sacc + jnp.abs(sdt)

    hacc, sacc = zero, zero
    for k in range(_NG):
        for d in descs[4 * k:4 * k + 4]:
            d.wait()
        hacc, sacc = lax.fori_loop(
            k * (_GCHUNK // _L), (k + 1) * (_GCHUNK // _L), chunk,
            (hacc, sacc))

    hsum = jnp.sum(hacc)
    ssum = jnp.sum(sacc)
    part_v[...] = jnp.where(lane == 0, hsum, jnp.where(lane == 1, ssum, 0.0))

    pltpu.sync_copy(t_v, t_hbm.at[pl.ds(base, _CHUNK)])
    pltpu.sync_copy(part_v, parts_hbm.at[wid])


@jax.jit
def _run(evp, evt, stloc, stdt, pt, pw, sidx, eidx):
    mesh = plsc.VectorSubcoreMesh(core_axis_name="c", subcore_axis_name="s",
                                  num_cores=_NC, num_subcores=_NS)
    f = pl.kernel(
        _sc_body,
        out_type=(
            jax.ShapeDtypeStruct((_B,), jnp.float32),
            jax.ShapeDtypeStruct((_NW, _L), jnp.float32),
        ),
        mesh=mesh,
        scratch_types=[
            pltpu.VMEM((_CHUNK,), jnp.int32),
            pltpu.VMEM((_CHUNK,), jnp.int32),
            pltpu.VMEM((_CHUNK,), jnp.float32),
            pltpu.VMEM((_CHUNK,), jnp.float32),
            pltpu.VMEM((_CHUNK,), jnp.float32),
            pltpu.VMEM((_CHUNK,), jnp.float32),
            pltpu.VMEM((_CHUNK,), jnp.float32),
            pltpu.VMEM((_CHUNK,), jnp.float32),
            pltpu.VMEM((_CHUNK,), jnp.float32),
            pltpu.VMEM((_NUM_STATION * 3,), jnp.float32),
            pltpu.VMEM((_NUM_STATION,), jnp.float32),
            pltpu.VMEM((_L,), jnp.float32),
            pltpu.SemaphoreType.DMA,
        ],
        compiler_params=pltpu.CompilerParams(needs_layout_passes=False,
                                             use_tc_tiling_on_sc=False),
        name="travel_time_sc",
    )
    t_flat, parts = f(evp, evt, stloc, stdt, pt, pw, sidx, eidx)
    t = t_flat.reshape(_B, 1)
    loss = (jnp.sum(parts[:, 0]) + _REG * jnp.sum(parts[:, 1])) / _B
    return t, loss


def kernel(event_loc_w, event_time_w, station_dt_w, station_loc, phase_time,
           phase_weight, station_index, event_index):
    return _run(
        event_loc_w.T.reshape(-1),
        event_time_w.reshape(-1),
        station_loc.reshape(-1),
        station_dt_w.reshape(-1),
        phase_time.reshape(-1),
        phase_weight.reshape(-1),
        station_index,
        event_index,
    )


# speedup vs baseline: 5.5991x; 1.0618x over previous
import jax
import jax.numpy as jnp
from jax import lax
from jax.experimental import pallas as pl
from jax.experimental.pallas import tpu as pltpu
from jax.experimental.pallas import tpu_sc as plsc

_NUM_EVENT = 100000
_NUM_STATION = 100
_B = 16384
_REG = 0.001

_NC = 2
_NS = 16
_NW = _NC * _NS
_CHUNK = _B // _NW
_GCHUNK = 512
_NG = _CHUNK // _GCHUNK
_L = 16


def _rsqrt(x):
    i = plsc.bitcast(x, jnp.int32)
    i = jnp.int32(0x5F3759DF) - (i >> 1)
    y = plsc.bitcast(i, jnp.float32)
    for _ in range(4):
        y = y * (1.5 - 0.5 * x * y * y)
    return y


def _sc_body(evp_hbm, stloc_hbm, stdt_hbm,
             pt_hbm, pw_hbm, sidx_hbm, eidx_hbm, t_hbm, parts_hbm,
             eidx_v, sidx_v, ex_v, ey_v, ez_v, et_v, pt_v, pw_v, t_v,
             stloc_v, stdt_v, part_v, sem):
    wid = lax.axis_index("s") * _NC + lax.axis_index("c")
    base = wid * _CHUNK

    pltpu.sync_copy(eidx_hbm.at[pl.ds(base, _CHUNK)], eidx_v)
    descs = []
    for k in range(_NG):
        isl = pl.ds(k * _GCHUNK, _GCHUNK)
        idx = eidx_v.at[isl]
        descs.append(pltpu.async_copy(
            evp_hbm.at[pl.ds(0, _NUM_EVENT)].at[idx], ex_v.at[isl], sem))
        descs.append(pltpu.async_copy(
            evp_hbm.at[pl.ds(_NUM_EVENT, _NUM_EVENT)].at[idx],
            ey_v.at[isl], sem))
        descs.append(pltpu.async_copy(
            evp_hbm.at[pl.ds(2 * _NUM_EVENT, _NUM_EVENT)].at[idx],
            ez_v.at[isl], sem))
        descs.append(pltpu.async_copy(
            evp_hbm.at[pl.ds(3 * _NUM_EVENT, _NUM_EVENT)].at[idx],
            et_v.at[isl], sem))

    pltpu.sync_copy(sidx_hbm.at[pl.ds(base, _CHUNK)], sidx_v)
    pltpu.sync_copy(pt_hbm.at[pl.ds(base, _CHUNK)], pt_v)
    pltpu.sync_copy(pw_hbm.at[pl.ds(base, _CHUNK)], pw_v)
    pltpu.sync_copy(stloc_hbm, stloc_v)
    pltpu.sync_copy(stdt_hbm, stdt_v)

    lane = lax.iota(jnp.int32, _L)
    zero = jnp.zeros((_L,), jnp.float32)

    def chunk(j, carry):
        hacc, sacc = carry
        off = j * _L
        sl = pl.ds(off, _L)
        sidx = sidx_v[sl]
        elx = ex_v[sl]
        ely = ey_v[sl]
        elz = ez_v[sl]
        et = et_v[sl]
        pt = pt_v[sl]
        pw = pw_v[sl]
        s3 = sidx * 3
        slx = plsc.load_gather(stloc_v, [s3])
        sly = plsc.load_gather(stloc_v, [s3 + 1])
        slz = plsc.load_gather(stloc_v, [s3 + 2])
        sdt = plsc.load_gather(stdt_v, [sidx])
        dx = elx - slx
        dy = ely - sly
        dz = elz - slz
        s = dx * dx + dy * dy + dz * dz
        dist = s * _rsqrt(s)
        t = et + dist + sdt
        t_v[sl] = t
        err = t - pt
        ae = jnp.abs(err)
        hub = jnp.where(ae < 1.0, 0.5 * err * err, ae - 0.5)
        return hacc + hub * pw, sacc + jnp.abs(sdt)

    hacc, sacc = zero, zero
    for k in range(_NG):
        for d in descs[4 * k:4 * k + 4]:
            d.wait()
        hacc, sacc = lax.fori_loop(
            k * (_GCHUNK // _L), (k + 1) * (_GCHUNK // _L), chunk,
            (hacc, sacc))

    hsum = jnp.sum(hacc)
    ssum = jnp.sum(sacc)
    part_v[...] = jnp.where(lane == 0, hsum, jnp.where(lane == 1, ssum, 0.0))

    pltpu.sync_copy(t_v, t_hbm.at[pl.ds(base, _CHUNK)])
    pltpu.sync_copy(part_v, parts_hbm.at[wid])


@jax.jit
def _run(evp, stloc, stdt, pt, pw, sidx, eidx):
    mesh = plsc.VectorSubcoreMesh(core_axis_name="c", subcore_axis_name="s",
                                  num_cores=_NC, num_subcores=_NS)
    f = pl.kernel(
        _sc_body,
        out_type=(
            jax.ShapeDtypeStruct((_B,), jnp.float32),
            jax.ShapeDtypeStruct((_NW, _L), jnp.float32),
        ),
        mesh=mesh,
        scratch_types=[
            pltpu.VMEM((_CHUNK,), jnp.int32),
            pltpu.VMEM((_CHUNK,), jnp.int32),
            pltpu.VMEM((_CHUNK,), jnp.float32),
            pltpu.VMEM((_CHUNK,), jnp.float32),
            pltpu.VMEM((_CHUNK,), jnp.float32),
            pltpu.VMEM((_CHUNK,), jnp.float32),
            pltpu.VMEM((_CHUNK,), jnp.float32),
            pltpu.VMEM((_CHUNK,), jnp.float32),
            pltpu.VMEM((_CHUNK,), jnp.float32),
            pltpu.VMEM((_NUM_STATION * 3,), jnp.float32),
            pltpu.VMEM((_NUM_STATION,), jnp.float32),
            pltpu.VMEM((_L,), jnp.float32),
            pltpu.SemaphoreType.DMA,
        ],
        compiler_params=pltpu.CompilerParams(needs_layout_passes=False,
                                             use_tc_tiling_on_sc=False),
        name="travel_time_sc",
    )
    t_flat, parts = f(evp, stloc, stdt, pt, pw, sidx, eidx)
    t = t_flat.reshape(_B, 1)
    lane_w = jnp.zeros((_L,), jnp.float32).at[0].set(1.0).at[1].set(_REG)
    loss = jnp.sum(parts * lane_w) / _B
    return t, loss


def kernel(event_loc_w, event_time_w, station_dt_w, station_loc, phase_time,
           phase_weight, station_index, event_index):
    return _run(
        jnp.concatenate([event_loc_w.T.reshape(-1),
                         event_time_w.reshape(-1)]),
        station_loc.reshape(-1),
        station_dt_w.reshape(-1),
        phase_time.reshape(-1),
        phase_weight.reshape(-1),
        station_index,
        event_index,
    )
